# Initial kernel scaffold; baseline (speedup 1.0000x reference)
import functools
import jax
import jax.numpy as jnp
from jax import lax
from jax.experimental import pallas as pl
from jax.experimental.pallas import tpu as pltpu
from jax.experimental.pallas import tpu_sc as plsc

_C = 26
_D = 100000
_F = 32
_B = 4096
_NW = 32
_BPW = _B // _NW
_RB = _BPW // 16


def _sc_body(cond_t, table, out, t_v, idx0_v, idx1_v, w_v, v0_b, v1_b,
             out_v, sem0, sem1):
    cid = lax.axis_index("c")
    sid = lax.axis_index("s")
    wid = sid * 2 + cid
    base = wid * _BPW

    def dim_body(i, _):
        pltpu.sync_copy(cond_t.at[i, pl.ds(base, _BPW)], t_v)
        for j in range(_RB):
            t = t_v[pl.ds(j * 16, 16)]
            ts = t * float(_D - 1)
            i0 = ts.astype(jnp.int32)
            w = ts - i0.astype(jnp.float32)
            g0 = i0 + i * _D
            idx0_v[pl.ds(j * 16, 16)] = g0
            idx1_v[pl.ds(j * 16, 16)] = g0 + 1
            w_v[pl.ds(j * 16, 16)] = w
        cp0 = pltpu.async_copy(table.at[idx0_v], v0_b, sem0)
        cp1 = pltpu.async_copy(table.at[idx1_v], v1_b, sem1)
        cp0.wait()
        cp1.wait()

        for rb in range(_RB):
            rows = rb * 16 + lax.iota(jnp.int32, 16)
            wv = w_v[pl.ds(rb * 16, 16)]

            def col_body(cc, _, rows=rows, wv=wv):
                cols = jnp.full((16,), cc, jnp.int32)
                a = plsc.load_gather(v0_b, [rows, cols])
                b = plsc.load_gather(v1_b, [rows, cols])
                o = a + wv * (b - a)
                plsc.store_scatter(out_v, [rows, cols], o)
                return 0

            lax.fori_loop(0, _F, col_body, 0)
        pltpu.sync_copy(out_v, out.at[pl.ds(base, _BPW), pl.ds(i * _F, _F)])
        return 0

    lax.fori_loop(0, _C, dim_body, 0)


_sc_kernel = functools.partial(
    pl.kernel,
    out_type=jax.ShapeDtypeStruct((_B, _C * _F), jnp.float32),
    mesh=plsc.VectorSubcoreMesh(core_axis_name="c", subcore_axis_name="s"),
    scratch_types=[
        pltpu.VMEM((_BPW,), jnp.float32),
        pltpu.VMEM((_BPW,), jnp.int32),
        pltpu.VMEM((_BPW,), jnp.int32),
        pltpu.VMEM((_BPW,), jnp.float32),
        pltpu.VMEM((_BPW, _F), jnp.float32),
        pltpu.VMEM((_BPW, _F), jnp.float32),
        pltpu.VMEM((_BPW, _F), jnp.float32),
        pltpu.SemaphoreType.DMA,
        pltpu.SemaphoreType.DMA,
    ],
)(_sc_body)


@jax.jit
def kernel(cond, lines):
    cond_t = cond.T
    table = lines.reshape(_C * _D, _F)
    return _sc_kernel(cond_t, table)


# speedup vs baseline: 1.0007x; 1.0007x over previous
import functools
import jax
import jax.numpy as jnp
from jax import lax
from jax.experimental import pallas as pl
from jax.experimental.pallas import tpu as pltpu
from jax.experimental.pallas import tpu_sc as plsc

_C = 26
_D = 100000
_F = 32
_B = 4096
_NW = 32
_BPW = _B // _NW
_RB = _BPW // 16


def _sc_body(cond_t, table, out, t_v, idx0_v, idx1_v, w_v, v0_b, v1_b,
             out_v, sem0, sem1):
    cid = lax.axis_index("c")
    sid = lax.axis_index("s")
    wid = sid * 2 + cid
    base = wid * _BPW

    def dim_body(i, _):
        pltpu.sync_copy(cond_t.at[i, pl.ds(base, _BPW)], t_v)
        for j in range(_RB):
            t = t_v[pl.ds(j * 16, 16)]
            ts = t * float(_D - 1)
            i0 = ts.astype(jnp.int32)
            w = ts - i0.astype(jnp.float32)
            g0 = i0 + i * _D
            idx0_v[pl.ds(j * 16, 16)] = g0
            idx1_v[pl.ds(j * 16, 16)] = g0 + 1
            w_v[pl.ds(j * 16, 16)] = w
        cp0 = pltpu.async_copy(table.at[idx0_v], v0_b, sem0)
        cp1 = pltpu.async_copy(table.at[idx1_v], v1_b, sem1)
        cp0.wait()
        cp1.wait()

        def blk_body(rb, _):
            wv16 = w_v[pl.ds(rb * 16, 16)]
            base_r = rb * 16
            for l in range(16):
                wv = jnp.full((16,), wv16[l], jnp.float32)
                r = base_r + l
                for h in range(_F // 16):
                    a = v0_b[r, pl.ds(h * 16, 16)]
                    b = v1_b[r, pl.ds(h * 16, 16)]
                    out_v[r, pl.ds(h * 16, 16)] = a + wv * (b - a)
            return 0

        lax.fori_loop(0, _RB, blk_body, 0)
        pltpu.sync_copy(out_v, out.at[pl.ds(base, _BPW), pl.ds(i * _F, _F)])
        return 0

    lax.fori_loop(0, _C, dim_body, 0)


_sc_kernel = functools.partial(
    pl.kernel,
    out_type=jax.ShapeDtypeStruct((_B, _C * _F), jnp.float32),
    mesh=plsc.VectorSubcoreMesh(core_axis_name="c", subcore_axis_name="s"),
    compiler_params=pltpu.CompilerParams(use_tc_tiling_on_sc=False),
    scratch_types=[
        pltpu.VMEM((_BPW,), jnp.float32),
        pltpu.VMEM((_BPW,), jnp.int32),
        pltpu.VMEM((_BPW,), jnp.int32),
        pltpu.VMEM((_BPW,), jnp.float32),
        pltpu.VMEM((_BPW, _F), jnp.float32),
        pltpu.VMEM((_BPW, _F), jnp.float32),
        pltpu.VMEM((_BPW, _F), jnp.float32),
        pltpu.SemaphoreType.DMA,
        pltpu.SemaphoreType.DMA,
    ],
)(_sc_body)


@jax.jit
def kernel(cond, lines):
    cond_t = cond.T
    table = lines.reshape(_C * _D, _F)
    return _sc_kernel(cond_t, table)


# speedup vs baseline: 1.0015x; 1.0008x over previous
import functools
import jax
import jax.numpy as jnp
from jax import lax
from jax.experimental import pallas as pl
from jax.experimental.pallas import tpu as pltpu
from jax.experimental.pallas import tpu_sc as plsc

_C = 26
_D = 100000
_F = 32
_B = 4096
_NW = 32
_BPW = _B // _NW
_RB = _BPW // 16


def _sc_body(cond_t, table, out, t_v, idx0_v, idx1_v, w_v, v0_b, v1_b,
             out_v, sem0, sem1):
    cid = lax.axis_index("c")
    sid = lax.axis_index("s")
    wid = sid * 2 + cid
    base = wid * _BPW

    def dim_body(i, _):
        pltpu.sync_copy(cond_t.at[i, pl.ds(base, _BPW)], t_v)
        for j in range(_RB):
            t = t_v[pl.ds(j * 16, 16)]
            ts = t * float(_D - 1)
            i0 = ts.astype(jnp.int32)
            w = ts - i0.astype(jnp.float32)
            idx0_v[pl.ds(j * 16, 16)] = i0
            idx1_v[pl.ds(j * 16, 16)] = i0 + 1
            w_v[pl.ds(j * 16, 16)] = w
        cp0 = pltpu.async_copy(table.at[i].at[idx0_v], v0_b, sem0)
        cp1 = pltpu.async_copy(table.at[i].at[idx1_v], v1_b, sem1)
        cp0.wait()
        cp1.wait()

        def blk_body(rb, _):
            wv16 = w_v[pl.ds(rb * 16, 16)]
            base_r = rb * 16
            for l in range(16):
                wv = jnp.full((16,), wv16[l], jnp.float32)
                r = base_r + l
                for h in range(_F // 16):
                    a = v0_b[r, pl.ds(h * 16, 16)]
                    b = v1_b[r, pl.ds(h * 16, 16)]
                    out_v[r, pl.ds(h * 16, 16)] = a + wv * (b - a)
            return 0

        lax.fori_loop(0, _RB, blk_body, 0)
        pltpu.sync_copy(out_v, out.at[pl.ds(base, _BPW), pl.ds(i * _F, _F)])
        return 0

    lax.fori_loop(0, _C, dim_body, 0)


_sc_kernel = functools.partial(
    pl.kernel,
    out_type=jax.ShapeDtypeStruct((_B, _C * _F), jnp.float32),
    mesh=plsc.VectorSubcoreMesh(core_axis_name="c", subcore_axis_name="s"),
    compiler_params=pltpu.CompilerParams(use_tc_tiling_on_sc=False),
    scratch_types=[
        pltpu.VMEM((_BPW,), jnp.float32),
        pltpu.VMEM((_BPW,), jnp.int32),
        pltpu.VMEM((_BPW,), jnp.int32),
        pltpu.VMEM((_BPW,), jnp.float32),
        pltpu.VMEM((_BPW, _F), jnp.float32),
        pltpu.VMEM((_BPW, _F), jnp.float32),
        pltpu.VMEM((_BPW, _F), jnp.float32),
        pltpu.SemaphoreType.DMA,
        pltpu.SemaphoreType.DMA,
    ],
)(_sc_body)


@jax.jit
def kernel(cond, lines):
    cond_t = cond.T
    return _sc_kernel(cond_t, lines)
